# baseline (speedup 1.0000x reference)
import functools

import jax
import jax.numpy as jnp
from jax import lax
from jax.experimental import pallas as pl
from jax.experimental.pallas import tpu as pltpu
from jax.experimental.pallas import tpu_sc as plsc

B, S, D = 16384, 100, 64
SD = S * D
NC, NS, L = 2, 16, 16
NW = NC * NS
RPW = B // NW
C = 8
NCHUNK = RPW // C


def _make_sc_add():
    mesh = plsc.VectorSubcoreMesh(core_axis_name="c", subcore_axis_name="s")

    @functools.partial(
        pl.kernel,
        mesh=mesh,
        out_type=jax.ShapeDtypeStruct((B, SD), jnp.float32),
        scratch_types=[
            pltpu.VMEM((SD,), jnp.float32),
            pltpu.VMEM((C, SD), jnp.float32),
        ],
    )
    def k(tok_hbm, emb_hbm, out_hbm, emb_v, buf):
        wid = lax.axis_index("s") * NC + lax.axis_index("c")
        base = wid * RPW
        pltpu.sync_copy(emb_hbm, emb_v)

        def chunk_body(g, carry):
            row0 = base + g * C
            pltpu.sync_copy(tok_hbm.at[pl.ds(row0, C)], buf)

            def jbody(j, carry2):
                col = pl.multiple_of(j * L, L)
                e = emb_v[pl.ds(col, L)]
                for r in range(C):
                    buf[r, pl.ds(col, L)] = buf[r, pl.ds(col, L)] + e
                return carry2

            lax.fori_loop(0, SD // L, jbody, 0)
            pltpu.sync_copy(buf, out_hbm.at[pl.ds(row0, C)])
            return carry

        lax.fori_loop(0, NCHUNK, chunk_body, 0)

    return k


_sc_add = _make_sc_add()


def kernel(tokens, id_embedding):
    tok2 = tokens.reshape(B, SD)
    emb = id_embedding.reshape(SD)
    out = _sc_add(tok2, emb)
    return out.reshape(B, S, D)


# speedup vs baseline: 1.3192x; 1.3192x over previous
import functools

import jax
import jax.numpy as jnp
from jax import lax
from jax.experimental import pallas as pl
from jax.experimental.pallas import tpu as pltpu
from jax.experimental.pallas import tpu_sc as plsc

B, S, D = 16384, 100, 64
SD = S * D
NC, NS, L = 2, 16, 16
NW = NC * NS
RPW = B // NW
C = 4
NCHUNK = RPW // C
NBUF = 2


def _make_sc_add():
    mesh = plsc.VectorSubcoreMesh(core_axis_name="c", subcore_axis_name="s")

    @functools.partial(
        pl.kernel,
        mesh=mesh,
        out_type=jax.ShapeDtypeStruct((B, SD), jnp.float32),
        scratch_types=[
            pltpu.VMEM((SD,), jnp.float32),
            pltpu.VMEM((NBUF, C, SD), jnp.float32),
            pltpu.VMEM((NBUF, C, SD), jnp.float32),
            pltpu.SemaphoreType.DMA,
            pltpu.SemaphoreType.DMA,
            pltpu.SemaphoreType.DMA,
            pltpu.SemaphoreType.DMA,
        ],
    )
    def k(tok_hbm, emb_hbm, out_hbm, emb_v, inb, outb, is0, is1, os0, os1):
        isems = [is0, is1]
        osems = [os0, os1]
        wid = lax.axis_index("s") * NC + lax.axis_index("c")
        base = wid * RPW
        pltpu.sync_copy(emb_hbm, emb_v)

        for b in range(NBUF):
            pltpu.async_copy(
                tok_hbm.at[pl.ds(base + b * C, C)], inb.at[b], isems[b]
            )

        def step(m, carry):
            for b in range(NBUF):
                g = m * NBUF + b
                row0 = base + g * C
                pltpu.make_async_copy(
                    tok_hbm.at[pl.ds(row0, C)], inb.at[b], isems[b]
                ).wait()

                @pl.when(g >= NBUF)
                def _():
                    pltpu.make_async_copy(
                        outb.at[b], out_hbm.at[pl.ds(row0, C)], osems[b]
                    ).wait()

                def jbody(j, c2):
                    col = pl.multiple_of(j * L, L)
                    e = emb_v[pl.ds(col, L)]
                    for r in range(C):
                        outb[b, r, pl.ds(col, L)] = inb[b, r, pl.ds(col, L)] + e
                    return c2

                lax.fori_loop(0, SD // L, jbody, 0)

                pltpu.async_copy(
                    outb.at[b], out_hbm.at[pl.ds(row0, C)], osems[b]
                )

                @pl.when(g + NBUF < NCHUNK)
                def _():
                    pltpu.async_copy(
                        tok_hbm.at[pl.ds(row0 + NBUF * C, C)],
                        inb.at[b],
                        isems[b],
                    )

            return carry

        lax.fori_loop(0, NCHUNK // NBUF, step, 0)

        for b in range(NBUF):
            row0 = base + (NCHUNK - NBUF + b) * C
            pltpu.make_async_copy(
                outb.at[b], out_hbm.at[pl.ds(row0, C)], osems[b]
            ).wait()

    return k


_sc_add = _make_sc_add()


def kernel(tokens, id_embedding):
    tok2 = tokens.reshape(B, SD)
    emb = id_embedding.reshape(SD)
    out = _sc_add(tok2, emb)
    return out.reshape(B, S, D)


# speedup vs baseline: 1.3595x; 1.0306x over previous
import functools

import jax
import jax.numpy as jnp
from jax import lax
from jax.experimental import pallas as pl
from jax.experimental.pallas import tpu as pltpu
from jax.experimental.pallas import tpu_sc as plsc

B, S, D = 16384, 100, 64
SD = S * D
NC, NS, L = 2, 16, 16
NW = NC * NS
RPW = B // NW
C = 4
NCHUNK = RPW // C
NBUF = 2


def _make_sc_add():
    mesh = plsc.VectorSubcoreMesh(core_axis_name="c", subcore_axis_name="s")

    @functools.partial(
        pl.kernel,
        mesh=mesh,
        out_type=jax.ShapeDtypeStruct((B, SD), jnp.float32),
        scratch_types=[
            pltpu.VMEM((SD,), jnp.float32),
            pltpu.VMEM((NBUF, C, SD), jnp.float32),
            pltpu.VMEM((NBUF, C, SD), jnp.float32),
            pltpu.SemaphoreType.DMA,
            pltpu.SemaphoreType.DMA,
            pltpu.SemaphoreType.DMA,
            pltpu.SemaphoreType.DMA,
        ],
    )
    def k(tok_hbm, emb_hbm, out_hbm, emb_v, inb, outb, is0, is1, os0, os1):
        isems = [is0, is1]
        osems = [os0, os1]
        wid = lax.axis_index("s") * NC + lax.axis_index("c")
        base = wid * RPW
        pltpu.sync_copy(emb_hbm, emb_v)

        for b in range(NBUF):
            pltpu.async_copy(
                tok_hbm.at[pl.ds(base + b * C, C)], inb.at[b], isems[b]
            )

        def step(m, carry):
            for b in range(NBUF):
                g = m * NBUF + b
                row0 = base + g * C
                pltpu.make_async_copy(
                    tok_hbm.at[pl.ds(row0, C)], inb.at[b], isems[b]
                ).wait()

                @pl.when(g >= NBUF)
                def _():
                    pltpu.make_async_copy(
                        outb.at[b], out_hbm.at[pl.ds(row0, C)], osems[b]
                    ).wait()

                @plsc.parallel_loop(0, SD // L, unroll=8)
                def _(j):
                    col = pl.multiple_of(j * L, L)
                    e = emb_v[pl.ds(col, L)]
                    for r in range(C):
                        outb[b, r, pl.ds(col, L)] = inb[b, r, pl.ds(col, L)] + e

                pltpu.async_copy(
                    outb.at[b], out_hbm.at[pl.ds(row0, C)], osems[b]
                )

                @pl.when(g + NBUF < NCHUNK)
                def _():
                    pltpu.async_copy(
                        tok_hbm.at[pl.ds(row0 + NBUF * C, C)],
                        inb.at[b],
                        isems[b],
                    )

            return carry

        lax.fori_loop(0, NCHUNK // NBUF, step, 0)

        for b in range(NBUF):
            row0 = base + (NCHUNK - NBUF + b) * C
            pltpu.make_async_copy(
                outb.at[b], out_hbm.at[pl.ds(row0, C)], osems[b]
            ).wait()

    return k


_sc_add = _make_sc_add()


def kernel(tokens, id_embedding):
    tok2 = tokens.reshape(B, SD)
    emb = id_embedding.reshape(SD)
    out = _sc_add(tok2, emb)
    return out.reshape(B, S, D)


# speedup vs baseline: 1.3597x; 1.0001x over previous
import functools

import jax
import jax.numpy as jnp
from jax import lax
from jax.experimental import pallas as pl
from jax.experimental.pallas import tpu as pltpu
from jax.experimental.pallas import tpu_sc as plsc

B, S, D = 16384, 100, 64
SD = S * D
NC, NS, L = 2, 16, 16
NW = NC * NS
RPW = B // NW
C = 4
NCHUNK = RPW // C
NBUF = 2


def _make_sc_add():
    mesh = plsc.VectorSubcoreMesh(
        core_axis_name="c", subcore_axis_name="s", num_cores=NC, num_subcores=NS
    )

    @functools.partial(
        pl.kernel,
        mesh=mesh,
        out_type=jax.ShapeDtypeStruct((B, SD), jnp.float32),
        scratch_types=[
            pltpu.VMEM((SD,), jnp.float32),
            pltpu.VMEM((NBUF, C, SD), jnp.float32),
            pltpu.VMEM((NBUF, C, SD), jnp.float32),
            pltpu.SemaphoreType.DMA,
            pltpu.SemaphoreType.DMA,
            pltpu.SemaphoreType.DMA,
            pltpu.SemaphoreType.DMA,
        ],
    )
    def k(tok_hbm, emb_hbm, out_hbm, emb_v, inb, outb, is0, is1, os0, os1):
        isems = [is0, is1]
        osems = [os0, os1]
        wid = lax.axis_index("s") * NC + lax.axis_index("c")
        base = wid * RPW
        pltpu.sync_copy(emb_hbm, emb_v)

        for b in range(NBUF):
            pltpu.async_copy(
                tok_hbm.at[pl.ds(base + b * C, C)], inb.at[b], isems[b]
            )

        def step(m, carry):
            for b in range(NBUF):
                g = m * NBUF + b
                row0 = base + g * C
                pltpu.make_async_copy(
                    tok_hbm.at[pl.ds(row0, C)], inb.at[b], isems[b]
                ).wait()

                @pl.when(g >= NBUF)
                def _():
                    pltpu.make_async_copy(
                        outb.at[b], out_hbm.at[pl.ds(row0, C)], osems[b]
                    ).wait()

                @plsc.parallel_loop(0, SD // L, unroll=8)
                def _(j):
                    col = pl.multiple_of(j * L, L)
                    e = emb_v[pl.ds(col, L)]
                    for r in range(C):
                        outb[b, r, pl.ds(col, L)] = inb[b, r, pl.ds(col, L)] + e

                pltpu.async_copy(
                    outb.at[b], out_hbm.at[pl.ds(row0, C)], osems[b]
                )

                @pl.when(g + NBUF < NCHUNK)
                def _():
                    pltpu.async_copy(
                        tok_hbm.at[pl.ds(row0 + NBUF * C, C)],
                        inb.at[b],
                        isems[b],
                    )

            return carry

        lax.fori_loop(0, NCHUNK // NBUF, step, 0)

        for b in range(NBUF):
            row0 = base + (NCHUNK - NBUF + b) * C
            pltpu.make_async_copy(
                outb.at[b], out_hbm.at[pl.ds(row0, C)], osems[b]
            ).wait()

    return k


_sc_add = _make_sc_add()


def kernel(tokens, id_embedding):
    tok2 = tokens.reshape(B, SD)
    emb = id_embedding.reshape(SD)
    out = _sc_add(tok2, emb)
    return out.reshape(B, S, D)


# speedup vs baseline: 4.5002x; 3.3097x over previous
import functools

import jax
import jax.numpy as jnp
from jax import lax
from jax.experimental import pallas as pl
from jax.experimental.pallas import tpu as pltpu
from jax.experimental.pallas import tpu_sc as plsc

B, S, D = 16384, 100, 64
NC, NS, L = 2, 16, 16
NW = NC * NS
DG = 16
CW = 1024
NLG = B // CW
NDG = D // DG
NQ = S * NDG * NLG
QPW = NQ // NW
NBUF = 2


def _make_sc_add():
    mesh = plsc.VectorSubcoreMesh(
        core_axis_name="c", subcore_axis_name="s", num_cores=NC, num_subcores=NS
    )

    @functools.partial(
        pl.kernel,
        mesh=mesh,
        out_type=jax.ShapeDtypeStruct((S, D, B), jnp.float32),
        scratch_types=[
            pltpu.VMEM((S * D,), jnp.float32),
            pltpu.VMEM((DG, CW), jnp.float32),
            pltpu.VMEM((DG, CW), jnp.float32),
            pltpu.VMEM((DG, CW), jnp.float32),
            pltpu.VMEM((DG, CW), jnp.float32),
            pltpu.SemaphoreType.DMA,
            pltpu.SemaphoreType.DMA,
            pltpu.SemaphoreType.DMA,
            pltpu.SemaphoreType.DMA,
        ],
    )
    def k(tok_hbm, emb_hbm, out_hbm, emb_v, in0, in1, ou0, ou1, is0, is1, os0, os1):
        inb = [in0, in1]
        oub = [ou0, ou1]
        isems = [is0, is1]
        osems = [os0, os1]
        wid = lax.axis_index("s") * NC + lax.axis_index("c")
        q0 = wid * QPW
        pltpu.sync_copy(emb_hbm, emb_v)

        def q_slices(q):
            s = q // (NDG * NLG)
            rem = q % (NDG * NLG)
            dg = rem // NLG
            lg = rem % NLG
            return s, dg * DG, lg * CW

        def in_copy(q, b):
            s, d0, c0 = q_slices(q)
            return pltpu.make_async_copy(
                tok_hbm.at[s, pl.ds(d0, DG), pl.ds(c0, CW)], inb[b], isems[b]
            )

        def out_copy(q, b):
            s, d0, c0 = q_slices(q)
            return pltpu.make_async_copy(
                oub[b], out_hbm.at[s, pl.ds(d0, DG), pl.ds(c0, CW)], osems[b]
            )

        for b in range(NBUF):
            in_copy(q0 + b, b).start()

        def step(m, carry):
            for b in range(NBUF):
                g = m * NBUF + b
                q = q0 + g
                in_copy(q, b).wait()

                @pl.when(g >= NBUF)
                def _():
                    out_copy(q, b).wait()

                s, d0, _ = q_slices(q)
                e0 = pl.multiple_of(s * D + d0, L)
                ev = emb_v[pl.ds(e0, L)]
                evs = [
                    jnp.full((L,), ev[r], dtype=jnp.float32)
                    for r in range(DG)
                ]

                @plsc.parallel_loop(0, CW // L, unroll=4)
                def _(i):
                    col = pl.multiple_of(i * L, L)
                    for r in range(DG):
                        oub[b][r, pl.ds(col, L)] = inb[b][r, pl.ds(col, L)] + evs[r]

                out_copy(q, b).start()

                @pl.when(g + NBUF < QPW)
                def _():
                    in_copy(q + NBUF, b).start()

            return carry

        lax.fori_loop(0, QPW // NBUF, step, 0)

        for b in range(NBUF):
            out_copy(q0 + QPW - NBUF + b, b).wait()

    return k


_sc_add = _make_sc_add()


def kernel(tokens, id_embedding):
    tok_t = jnp.transpose(tokens, (1, 2, 0))
    emb = id_embedding.reshape(S * D)
    out_t = _sc_add(tok_t, emb)
    return jnp.transpose(out_t, (2, 0, 1))
